# baseline (speedup 1.0000x reference)
import functools

import jax
import jax.numpy as jnp
from jax.experimental import pallas as pl
from jax.experimental.pallas import tpu as pltpu

B = 8
C = 3
H = 352
N = 484
NPAD = 512
D = 768
K = 9
NEG = -3e38


def _rb(v):
    return v.astype(jnp.bfloat16).astype(jnp.float32)


def _band_matmul(src_b, band_ref):
    acc = jnp.zeros((H, C * H), jnp.float32)
    for ci in range(C):
        for dh in range(3):
            lhs = src_b[ci, dh:dh + H, :]
            rhs = band_ref[ci * 3 + dh]
            acc = acc + jnp.dot(lhs, rhs, preferred_element_type=jnp.float32)
    return acc


def _conv_patch_kernel(b0_ref, b1_ref, band0_ref, band1_ref, x_ref, amp_ref,
                       p_ref, xs, hs):
    @pl.when(pl.program_id(0) == 0)
    def _init():
        xs[...] = jnp.zeros_like(xs)
        hs[...] = jnp.zeros_like(hs)

    xs[:, 1:H + 1, 1:H + 1] = x_ref[0]
    xp = xs[...]
    y0 = _band_matmul(xp.astype(jnp.bfloat16), band0_ref)
    for co in range(C):
        hco = jnp.maximum(y0[:, co * H:(co + 1) * H] + b0_ref[co], 0.0)
        hs[co, 1:H + 1, 1:H + 1] = hco
    y1 = _band_matmul(hs[...].astype(jnp.bfloat16), band1_ref)
    prs = []
    for co in range(C):
        pr = y1[:, co * H:(co + 1) * H] + b1_ref[co]
        amp_ref[0, co] = pr * xp[co, 1:H + 1, 1:H + 1]
        prs.append(pr)
    v = jnp.stack(prs)
    v = v.reshape(C, 22, 16, 22, 16)
    v = jnp.transpose(v, (1, 3, 0, 2, 4))
    v = v.reshape(N, D)
    p_ref[0] = jnp.concatenate([v, jnp.zeros((NPAD - N, D), jnp.float32)], axis=0)


def _make_band(W):
    jp = jnp.arange(H + 2)
    j = jnp.arange(H)
    dw = jp[:, None] - j[None, :]
    valid = (dw >= 0) & (dw <= 2)
    dwc = jnp.clip(dw, 0, 2)
    vals = W[:, :, :, dwc]
    vals = jnp.where(valid[None, None, None], vals, 0.0)
    vals = vals.transpose(1, 2, 3, 0, 4)
    return vals.reshape(C * 3, H + 2, C * H).astype(jnp.bfloat16)


def _vig_kernel(p_ref, we_ref, be_ref, wgt_ref, wgb_ref, bg_ref, out_ref):
    pb = p_ref[0].astype(jnp.bfloat16)
    feat = jnp.dot(pb, we_ref[...], preferred_element_type=jnp.float32)
    feat = feat + be_ref[...]
    sq = jnp.sum(feat * feat, axis=1, keepdims=True)
    fb = feat.astype(jnp.bfloat16)
    gram = jax.lax.dot_general(fb, fb, (((1,), (1,)), ((), ())),
                               preferred_element_type=jnp.float32)
    dist = sq + sq.T - 2.0 * gram
    col = jax.lax.broadcasted_iota(jnp.int32, (NPAD, NPAD), 1)
    dist = jnp.where(col < N, dist, jnp.inf)
    maxrel = jnp.full((NPAD, D), NEG, jnp.float32)
    for _ in range(K):
        rowmin = jnp.min(dist, axis=1, keepdims=True)
        cand = jnp.where(dist == rowmin, col, NPAD)
        sel = jnp.min(cand, axis=1, keepdims=True)
        onehot = (col == sel)
        nb = jnp.dot(onehot.astype(jnp.bfloat16), fb,
                     preferred_element_type=jnp.float32)
        maxrel = jnp.maximum(maxrel, nb)
        dist = jnp.where(onehot, jnp.inf, dist)
    maxrel = maxrel - feat
    hh = jnp.dot(fb, wgt_ref[...], preferred_element_type=jnp.float32)
    hh = hh + jnp.dot(maxrel.astype(jnp.bfloat16), wgb_ref[...],
                      preferred_element_type=jnp.float32)
    hh = hh + bg_ref[...]
    out_ref[0] = feat + jnp.maximum(hh, 0.0)


def kernel(x, W0, b0, W1, b1, We, be, Wg, bg):
    band0 = _make_band(W0)
    band1 = _make_band(W1)

    amp_src, p = pl.pallas_call(
        _conv_patch_kernel,
        grid=(B,),
        in_specs=[
            pl.BlockSpec(memory_space=pltpu.SMEM),
            pl.BlockSpec(memory_space=pltpu.SMEM),
            pl.BlockSpec((C * 3, H + 2, C * H), lambda i: (0, 0, 0)),
            pl.BlockSpec((C * 3, H + 2, C * H), lambda i: (0, 0, 0)),
            pl.BlockSpec((1, C, H, H), lambda i: (i, 0, 0, 0)),
        ],
        out_specs=[
            pl.BlockSpec((1, C, H, H), lambda i: (i, 0, 0, 0)),
            pl.BlockSpec((1, NPAD, D), lambda i: (i, 0, 0)),
        ],
        out_shape=[
            jax.ShapeDtypeStruct((B, C, H, H), jnp.float32),
            jax.ShapeDtypeStruct((B, NPAD, D), jnp.float32),
        ],
        scratch_shapes=[
            pltpu.VMEM((C, H + 2, H + 2), jnp.float32),
            pltpu.VMEM((C, H + 2, H + 2), jnp.float32),
        ],
    )(b0, b1, band0, band1, x)

    out = pl.pallas_call(
        _vig_kernel,
        grid=(B,),
        in_specs=[
            pl.BlockSpec((1, NPAD, D), lambda i: (i, 0, 0)),
            pl.BlockSpec((D, D), lambda i: (0, 0)),
            pl.BlockSpec((1, D), lambda i: (0, 0)),
            pl.BlockSpec((D, D), lambda i: (0, 0)),
            pl.BlockSpec((D, D), lambda i: (0, 0)),
            pl.BlockSpec((1, D), lambda i: (0, 0)),
        ],
        out_specs=pl.BlockSpec((1, NPAD, D), lambda i: (i, 0, 0)),
        out_shape=jax.ShapeDtypeStruct((B, NPAD, D), jnp.float32),
    )(p, We.astype(jnp.bfloat16), be.reshape(1, D),
      Wg[:D].astype(jnp.bfloat16), Wg[D:].astype(jnp.bfloat16),
      bg.reshape(1, D))

    amp_low = out[:, :N, :]
    return (amp_src, amp_low)


# speedup vs baseline: 40.9528x; 40.9528x over previous
import functools

import jax
import jax.numpy as jnp
from jax.experimental import pallas as pl
from jax.experimental.pallas import tpu as pltpu

B = 8
C = 3
H = 352
N = 484
NPAD = 512
D = 768
K = 9
NEG = -3e38


def _rb(v):
    return v.astype(jnp.bfloat16).astype(jnp.float32)


def _band_matmul(src_b, band_ref):
    acc = jnp.zeros((H, C * H), jnp.float32)
    for ci in range(C):
        for dh in range(3):
            lhs = src_b[ci, dh:dh + H, :]
            rhs = band_ref[ci * 3 + dh]
            acc = acc + jnp.dot(lhs, rhs, preferred_element_type=jnp.float32)
    return acc


def _conv_patch_kernel(b0_ref, b1_ref, band0_ref, band1_ref, x_ref, amp_ref,
                       p_ref, xs, hs):
    @pl.when(pl.program_id(0) == 0)
    def _init():
        xs[...] = jnp.zeros_like(xs)
        hs[...] = jnp.zeros_like(hs)

    xs[:, 1:H + 1, 1:H + 1] = x_ref[0]
    xp = xs[...]
    y0 = _band_matmul(xp.astype(jnp.bfloat16), band0_ref)
    for co in range(C):
        hco = jnp.maximum(y0[:, co * H:(co + 1) * H] + b0_ref[co], 0.0)
        hs[co, 1:H + 1, 1:H + 1] = hco
    y1 = _band_matmul(hs[...].astype(jnp.bfloat16), band1_ref)
    prs = []
    for co in range(C):
        pr = y1[:, co * H:(co + 1) * H] + b1_ref[co]
        amp_ref[0, co] = pr * xp[co, 1:H + 1, 1:H + 1]
        prs.append(pr)
    v = jnp.stack(prs)
    v = v.reshape(C, 22, 16, 22, 16)
    v = jnp.transpose(v, (1, 3, 0, 2, 4))
    v = v.reshape(N, D)
    p_ref[0] = jnp.concatenate([v, jnp.zeros((NPAD - N, D), jnp.float32)], axis=0)


def _make_band(W):
    jp = jnp.arange(H + 2)
    j = jnp.arange(H)
    diff = jp[:, None] - j[None, :]
    vals = jnp.zeros((C, C, 3, H + 2, H), jnp.float32)
    for dw in range(3):
        mask = (diff == dw).astype(jnp.float32)
        vals = vals + W[:, :, :, dw][..., None, None] * mask
    vals = vals.transpose(1, 2, 3, 0, 4)
    return vals.reshape(C * 3, H + 2, C * H).astype(jnp.bfloat16)


def _vig_kernel(p_ref, we_ref, be_ref, wgt_ref, wgb_ref, bg_ref, out_ref):
    pb = p_ref[0].astype(jnp.bfloat16)
    feat = jnp.dot(pb, we_ref[...], preferred_element_type=jnp.float32)
    feat = feat + be_ref[...]
    sq = jnp.sum(feat * feat, axis=1, keepdims=True)
    fb = feat.astype(jnp.bfloat16)
    gram = jax.lax.dot_general(fb, fb, (((1,), (1,)), ((), ())),
                               preferred_element_type=jnp.float32)
    dist = sq + sq.T - 2.0 * gram
    col = jax.lax.broadcasted_iota(jnp.int32, (NPAD, NPAD), 1)
    dist = jnp.where(col < N, dist, jnp.inf)
    maxrel = jnp.full((NPAD, D), NEG, jnp.float32)
    for _ in range(K):
        rowmin = jnp.min(dist, axis=1, keepdims=True)
        cand = jnp.where(dist == rowmin, col, NPAD)
        sel = jnp.min(cand, axis=1, keepdims=True)
        onehot = (col == sel)
        nb = jnp.dot(onehot.astype(jnp.bfloat16), fb,
                     preferred_element_type=jnp.float32)
        maxrel = jnp.maximum(maxrel, nb)
        dist = jnp.where(onehot, jnp.inf, dist)
    maxrel = maxrel - feat
    hh = jnp.dot(fb, wgt_ref[...], preferred_element_type=jnp.float32)
    hh = hh + jnp.dot(maxrel.astype(jnp.bfloat16), wgb_ref[...],
                      preferred_element_type=jnp.float32)
    hh = hh + bg_ref[...]
    out_ref[0] = feat + jnp.maximum(hh, 0.0)


def kernel(x, W0, b0, W1, b1, We, be, Wg, bg):
    band0 = _make_band(W0)
    band1 = _make_band(W1)

    amp_src, p = pl.pallas_call(
        _conv_patch_kernel,
        grid=(B,),
        in_specs=[
            pl.BlockSpec(memory_space=pltpu.SMEM),
            pl.BlockSpec(memory_space=pltpu.SMEM),
            pl.BlockSpec((C * 3, H + 2, C * H), lambda i: (0, 0, 0)),
            pl.BlockSpec((C * 3, H + 2, C * H), lambda i: (0, 0, 0)),
            pl.BlockSpec((1, C, H, H), lambda i: (i, 0, 0, 0)),
        ],
        out_specs=[
            pl.BlockSpec((1, C, H, H), lambda i: (i, 0, 0, 0)),
            pl.BlockSpec((1, NPAD, D), lambda i: (i, 0, 0)),
        ],
        out_shape=[
            jax.ShapeDtypeStruct((B, C, H, H), jnp.float32),
            jax.ShapeDtypeStruct((B, NPAD, D), jnp.float32),
        ],
        scratch_shapes=[
            pltpu.VMEM((C, H + 2, H + 2), jnp.float32),
            pltpu.VMEM((C, H + 2, H + 2), jnp.float32),
        ],
    )(b0, b1, band0, band1, x)

    out = pl.pallas_call(
        _vig_kernel,
        grid=(B,),
        in_specs=[
            pl.BlockSpec((1, NPAD, D), lambda i: (i, 0, 0)),
            pl.BlockSpec((D, D), lambda i: (0, 0)),
            pl.BlockSpec((1, D), lambda i: (0, 0)),
            pl.BlockSpec((D, D), lambda i: (0, 0)),
            pl.BlockSpec((D, D), lambda i: (0, 0)),
            pl.BlockSpec((1, D), lambda i: (0, 0)),
        ],
        out_specs=pl.BlockSpec((1, NPAD, D), lambda i: (i, 0, 0)),
        out_shape=jax.ShapeDtypeStruct((B, NPAD, D), jnp.float32),
    )(p, We.astype(jnp.bfloat16), be.reshape(1, D),
      Wg[:D].astype(jnp.bfloat16), Wg[D:].astype(jnp.bfloat16),
      bg.reshape(1, D))

    amp_low = out[:, :N, :]
    return (amp_src, amp_low)
